# baseline (speedup 1.0000x reference)
import functools
import math

import jax
import jax.numpy as jnp
from jax import lax
from jax.experimental import pallas as pl
from jax.experimental.pallas import tpu as pltpu
from jax.experimental.pallas import tpu_sc as plsc

D_MODEL = 64
SCALE = math.sqrt(D_MODEL)

NUM_CORES = 2
NUM_SUBCORES = 16
NUM_WORKERS = NUM_CORES * NUM_SUBCORES

GROUP = 128


def _scale_table_body(t_ref, o_ref):
    o_ref[...] = t_ref[...] * SCALE


def _scale_table(table):
    vocab, d = table.shape
    rows_per_block = 2000
    grid = vocab // rows_per_block
    return pl.pallas_call(
        _scale_table_body,
        out_shape=jax.ShapeDtypeStruct((vocab, d), jnp.float32),
        grid=(grid,),
        in_specs=[pl.BlockSpec((rows_per_block, d), lambda i: (i, 0))],
        out_specs=pl.BlockSpec((rows_per_block, d), lambda i: (i, 0)),
    )(table)


@functools.cache
def _make_gather(nb, nt, vocab, d):
    b_per_w = nb // NUM_WORKERS
    assert b_per_w % 2 == 0
    g0 = min(nt, GROUP)
    g1 = nt - g0
    d2 = 2 * d
    mesh = plsc.VectorSubcoreMesh(
        core_axis_name="c",
        subcore_axis_name="s",
        num_cores=NUM_CORES,
        num_subcores=NUM_SUBCORES,
    )

    @functools.partial(
        pl.kernel,
        out_type=jax.ShapeDtypeStruct((nb * nt, d2), jnp.float32),
        mesh=mesh,
        scratch_types=[
            pltpu.VMEM((b_per_w, g0), jnp.int32),
            pltpu.VMEM((b_per_w, g1), jnp.int32),
            pltpu.VMEM((2, nt, d), jnp.float32),
            pltpu.SemaphoreType.DMA,
            pltpu.SemaphoreType.DMA,
            pltpu.SemaphoreType.DMA,
            pltpu.SemaphoreType.DMA,
        ],
        compiler_params=pltpu.CompilerParams(use_tc_tiling_on_sc=False),
    )
    def gather_kernel(table_hbm, idxa_hbm, idxb_hbm, out_hbm, idxa_v, idxb_v,
                      rows_v, sg0, sg1, so0, so1):
        wid = lax.axis_index("s") * NUM_CORES + lax.axis_index("c")
        b_base = wid * b_per_w
        pltpu.sync_copy(idxa_hbm.at[pl.ds(b_base, b_per_w)], idxa_v)
        pltpu.sync_copy(idxb_hbm.at[pl.ds(b_base, b_per_w)], idxb_v)

        def gcps(j, buf, sem):
            return (
                pltpu.make_async_copy(
                    table_hbm.at[idxa_v.at[j]],
                    rows_v.at[buf, pl.ds(0, g0)],
                    sem,
                ),
                pltpu.make_async_copy(
                    table_hbm.at[idxb_v.at[j]],
                    rows_v.at[buf, pl.ds(g0, g1)],
                    sem,
                ),
            )

        def gstart(j, buf, sem):
            a, bb = gcps(j, buf, sem)
            a.start()
            bb.start()

        def gwait(j, buf, sem):
            a, bb = gcps(j, buf, sem)
            a.wait()
            bb.wait()

        def scale_buf(buf):
            def srow(i, carry):
                r = 4 * i
                for k in range(4):
                    for c in range(0, d, 16):
                        rows_v[buf, r + k, pl.ds(c, 16)] = (
                            rows_v[buf, r + k, pl.ds(c, 16)] * SCALE
                        )
                return carry

            lax.fori_loop(0, nt // 4, srow, 0)

        def ocp(j, buf, sem):
            return pltpu.make_async_copy(
                rows_v.at[buf],
                out_hbm.at[pl.ds((b_base + j) * nt, nt), pl.ds(0, d)],
                sem,
            )

        gstart(0, 0, sg0)

        def body(k, carry):
            j0 = 2 * k
            j1 = j0 + 1

            @pl.when(k > 0)
            def _():
                ocp(j0 - 1, 1, so1).wait()

            gstart(j1, 1, sg1)
            gwait(j0, 0, sg0)
            scale_buf(0)
            ocp(j0, 0, so0).start()
            gwait(j1, 1, sg1)
            scale_buf(1)
            ocp(j0, 0, so0).wait()
            ocp(j1, 1, so1).start()

            @pl.when(k < b_per_w // 2 - 1)
            def _():
                gstart(j0 + 2, 0, sg0)

            return carry

        lax.fori_loop(0, b_per_w // 2, body, 0)
        ocp(b_per_w - 1, 1, so1).wait()

    return gather_kernel


def kernel(x, table):
    b, t = x.shape
    vocab, d = table.shape
    idx = x.astype(jnp.int32)
    g0 = min(t, GROUP)
    idx_a = lax.slice(idx, (0, 0), (b, g0))
    idx_b = lax.slice(idx, (0, g0), (b, t))
    out2 = _make_gather(b, t, vocab, d)(table, idx_a, idx_b)
    out3 = out2.reshape(b, t, 2 * d)
    return lax.slice(out3, (0, 0, 0), (b, t, d))


# speedup vs baseline: 1.0576x; 1.0576x over previous
import functools
import math

import jax
import jax.numpy as jnp
from jax import lax
from jax.experimental import pallas as pl
from jax.experimental.pallas import tpu as pltpu
from jax.experimental.pallas import tpu_sc as plsc

D_MODEL = 64
SCALE = math.sqrt(D_MODEL)

NUM_CORES = 2
NUM_SUBCORES = 16
NUM_WORKERS = NUM_CORES * NUM_SUBCORES

GROUP = 128


def _scale_table_body(t_ref, o_ref):
    o_ref[...] = t_ref[...] * SCALE


def _scale_table(table):
    vocab, d = table.shape
    rows_per_block = 2000
    grid = vocab // rows_per_block
    return pl.pallas_call(
        _scale_table_body,
        out_shape=jax.ShapeDtypeStruct((vocab, d), jnp.float32),
        grid=(grid,),
        in_specs=[pl.BlockSpec((rows_per_block, d), lambda i: (i, 0))],
        out_specs=pl.BlockSpec((rows_per_block, d), lambda i: (i, 0)),
    )(table)


@functools.cache
def _make_gather(nb, nt, vocab, d):
    b_per_w = nb // NUM_WORKERS
    assert b_per_w % 2 == 0
    g0 = min(nt, GROUP)
    g1 = nt - g0
    d2 = 2 * d
    mesh = plsc.VectorSubcoreMesh(
        core_axis_name="c",
        subcore_axis_name="s",
        num_cores=NUM_CORES,
        num_subcores=NUM_SUBCORES,
    )

    @functools.partial(
        pl.kernel,
        out_type=jax.ShapeDtypeStruct((nb * nt, d2), jnp.float32),
        mesh=mesh,
        scratch_types=[
            pltpu.VMEM((b_per_w, g0), jnp.int32),
            pltpu.VMEM((b_per_w, g1), jnp.int32),
            pltpu.VMEM((4, nt, d), jnp.float32),
            pltpu.SemaphoreType.DMA,
            pltpu.SemaphoreType.DMA,
            pltpu.SemaphoreType.DMA,
            pltpu.SemaphoreType.DMA,
            pltpu.SemaphoreType.DMA,
            pltpu.SemaphoreType.DMA,
            pltpu.SemaphoreType.DMA,
            pltpu.SemaphoreType.DMA,
        ],
        compiler_params=pltpu.CompilerParams(use_tc_tiling_on_sc=False),
    )
    def gather_kernel(table_hbm, idxa_hbm, idxb_hbm, out_hbm, idxa_v, idxb_v,
                      rows_v, sg0, sg1, sg2, sg3, so0, so1, so2, so3):
        sg = (sg0, sg1, sg2, sg3)
        so = (so0, so1, so2, so3)
        wid = lax.axis_index("s") * NUM_CORES + lax.axis_index("c")
        b_base = wid * b_per_w
        pltpu.sync_copy(idxa_hbm.at[pl.ds(b_base, b_per_w)], idxa_v)
        pltpu.sync_copy(idxb_hbm.at[pl.ds(b_base, b_per_w)], idxb_v)

        def gcps(j, buf, sem):
            return (
                pltpu.make_async_copy(
                    table_hbm.at[idxa_v.at[j]],
                    rows_v.at[buf, pl.ds(0, g0)],
                    sem,
                ),
                pltpu.make_async_copy(
                    table_hbm.at[idxb_v.at[j]],
                    rows_v.at[buf, pl.ds(g0, g1)],
                    sem,
                ),
            )

        def gstart(j, buf, sem):
            a, bb = gcps(j, buf, sem)
            a.start()
            bb.start()

        def gwait(j, buf, sem):
            a, bb = gcps(j, buf, sem)
            a.wait()
            bb.wait()

        def scale_buf(buf):
            def srow(i, carry):
                r = 4 * i
                for k in range(4):
                    for c in range(0, d, 16):
                        rows_v[buf, r + k, pl.ds(c, 16)] = (
                            rows_v[buf, r + k, pl.ds(c, 16)] * SCALE
                        )
                return carry

            lax.fori_loop(0, nt // 4, srow, 0)

        def ocp(j, buf, sem):
            return pltpu.make_async_copy(
                rows_v.at[buf],
                out_hbm.at[pl.ds((b_base + j) * nt, nt), pl.ds(0, d)],
                sem,
            )

        gstart(0, 0, sg[0])
        gstart(1, 1, sg[1])
        gstart(2, 2, sg[2])
        n_outer = b_per_w // 4

        def body(k, carry):
            for i in range(4):
                j = 4 * k + i
                gwait(j, i, sg[i])
                scale_buf(i)
                ocp(j, i, so[i]).start()
                prev_buf = (i - 1) % 4
                if i == 0:
                    @pl.when(k > 0)
                    def _():
                        ocp(j - 1, prev_buf, so[prev_buf]).wait()

                    gstart(j + 3, prev_buf, sg[prev_buf])
                else:
                    ocp(j - 1, prev_buf, so[prev_buf]).wait()

                    @pl.when(k < n_outer - 1)
                    def _():
                        gstart(j + 3, prev_buf, sg[prev_buf])

            return carry

        lax.fori_loop(0, n_outer, body, 0)
        ocp(b_per_w - 1, 3, so[3]).wait()

    return gather_kernel


def kernel(x, table):
    b, t = x.shape
    vocab, d = table.shape
    idx = x.astype(jnp.int32)
    g0 = min(t, GROUP)
    idx_a = lax.slice(idx, (0, 0), (b, g0))
    idx_b = lax.slice(idx, (0, g0), (b, t))
    out2 = _make_gather(b, t, vocab, d)(table, idx_a, idx_b)
    out3 = out2.reshape(b, t, 2 * d)
    return lax.slice(out3, (0, 0, 0), (b, t, d))


# speedup vs baseline: 1.0581x; 1.0005x over previous
import functools
import math

import jax
import jax.numpy as jnp
from jax import lax
from jax.experimental import pallas as pl
from jax.experimental.pallas import tpu as pltpu
from jax.experimental.pallas import tpu_sc as plsc

D_MODEL = 64
SCALE = math.sqrt(D_MODEL)

NUM_CORES = 2
NUM_SUBCORES = 16
NUM_WORKERS = NUM_CORES * NUM_SUBCORES

GROUP = 128


def _scale_table_body(t_ref, o_ref):
    o_ref[...] = t_ref[...] * SCALE


def _scale_table(table):
    vocab, d = table.shape
    rows_per_block = 2000
    grid = vocab // rows_per_block
    return pl.pallas_call(
        _scale_table_body,
        out_shape=jax.ShapeDtypeStruct((vocab, d), jnp.float32),
        grid=(grid,),
        in_specs=[pl.BlockSpec((rows_per_block, d), lambda i: (i, 0))],
        out_specs=pl.BlockSpec((rows_per_block, d), lambda i: (i, 0)),
    )(table)


@functools.cache
def _make_gather(nb, nt, vocab, d):
    b_per_w = nb // NUM_WORKERS
    assert b_per_w % 4 == 0
    g0 = 104
    g1 = nt - g0
    assert g1 <= GROUP and g0 % 8 == 0 and g1 % 8 == 0
    d2 = 2 * d
    n_slots = 2 * b_per_w
    DEPTH = 8
    mesh = plsc.VectorSubcoreMesh(
        core_axis_name="c",
        subcore_axis_name="s",
        num_cores=NUM_CORES,
        num_subcores=NUM_SUBCORES,
    )

    @functools.partial(
        pl.kernel,
        out_type=jax.ShapeDtypeStruct((nb * nt, d2), jnp.float32),
        mesh=mesh,
        scratch_types=[
            pltpu.VMEM((b_per_w, g0), jnp.int32),
            pltpu.VMEM((b_per_w, g1), jnp.int32),
            pltpu.VMEM((DEPTH, g0, d), jnp.float32),
        ]
        + [pltpu.SemaphoreType.DMA] * (2 * DEPTH),
        compiler_params=pltpu.CompilerParams(use_tc_tiling_on_sc=False),
    )
    def gather_kernel(table_hbm, idxa_hbm, idxb_hbm, out_hbm, idxa_v, idxb_v,
                      rows_v, *sems):
        sg = sems[:DEPTH]
        so = sems[DEPTH:]
        wid = lax.axis_index("s") * NUM_CORES + lax.axis_index("c")
        b_base = wid * b_per_w
        pltpu.sync_copy(idxa_hbm.at[pl.ds(b_base, b_per_w)], idxa_v)
        pltpu.sync_copy(idxb_hbm.at[pl.ds(b_base, b_per_w)], idxb_v)

        def gcp(j, half, buf, sem):
            idx_ref = idxa_v if half == 0 else idxb_v
            glen = g0 if half == 0 else g1
            return pltpu.make_async_copy(
                table_hbm.at[idx_ref.at[j // 2]],
                rows_v.at[buf, pl.ds(0, glen)],
                sem,
            )

        def scale_buf(buf, half):
            glen = g0 if half == 0 else g1
            def srow(i, carry):
                r = 4 * i
                for k in range(4):
                    for c in range(0, d, 16):
                        rows_v[buf, r + k, pl.ds(c, 16)] = (
                            rows_v[buf, r + k, pl.ds(c, 16)] * SCALE
                        )
                return carry

            lax.fori_loop(0, glen // 4, srow, 0)

        def ocp(j, half, buf, sem):
            glen = g0 if half == 0 else g1
            row0 = (b_base + j // 2) * nt + (0 if half == 0 else g0)
            return pltpu.make_async_copy(
                rows_v.at[buf, pl.ds(0, glen)],
                out_hbm.at[pl.ds(row0, glen), pl.ds(0, d)],
                sem,
            )

        for p in range(DEPTH - 1):
            gcp(p, p % 2, p, sg[p]).start()
        n_outer = n_slots // DEPTH

        def body(k, carry):
            for i in range(DEPTH):
                j = DEPTH * k + i
                half = i % 2
                gcp(j, half, i, sg[i]).wait()
                scale_buf(i, half)
                ocp(j, half, i, so[i]).start()
                pb = (i - 1) % DEPTH
                phalf = (i - 1) % 2
                if i == 0:
                    @pl.when(k > 0)
                    def _():
                        ocp(j - 1, phalf, pb, so[pb]).wait()

                    gcp(j + DEPTH - 1, phalf, pb, sg[pb]).start()
                else:
                    ocp(j - 1, phalf, pb, so[pb]).wait()

                    @pl.when(k < n_outer - 1)
                    def _():
                        gcp(j + DEPTH - 1, phalf, pb, sg[pb]).start()

            return carry

        lax.fori_loop(0, n_outer, body, 0)
        ocp(n_slots - 1, 1, DEPTH - 1, so[DEPTH - 1]).wait()

    return gather_kernel


def kernel(x, table):
    b, t = x.shape
    vocab, d = table.shape
    idx = x.astype(jnp.int32)
    g0 = 104
    idx_a = lax.slice(idx, (0, 0), (b, g0))
    idx_b = lax.slice(idx, (0, g0), (b, t))
    out2 = _make_gather(b, t, vocab, d)(table, idx_a, idx_b)
    out3 = out2.reshape(b, t, 2 * d)
    return lax.slice(out3, (0, 0, 0), (b, t, d))
